# baseline (speedup 1.0000x reference)
import functools

import jax
import jax.numpy as jnp
from jax import lax
from jax.experimental import pallas as pl
from jax.experimental.pallas import tpu as pltpu
from jax.experimental.pallas import tpu_sc as plsc

N = 10000
E = 320000
D = 128
KS = (5000, 2500, 1250)

NC = 2
NS = 16
NW = NC * NS
CH = 128
GPT = 10
CPT = GPT * 8
EPAD = NW * CPT * CH - E
XP = N + 16
STRIPE = 624

NPAD = 10240
SROWS = NPAD // 128
RB = 2000
GA = N // RB
IMIN = -2147483648


def _sc_scatter_body(x_hbm, src_hbm, dst_hbm, out_hbm,
                     srcg, dstg, rows_a, rows_b, acc, sem_i, sem_a, sem_b):
    cid = lax.axis_index("c")
    sid = lax.axis_index("s")
    wid = sid * NC + cid
    g0 = wid * GPT

    pltpu.async_copy(src_hbm.at[g0], srcg.at[0], sem_i)
    pltpu.async_copy(dst_hbm.at[g0], dstg.at[0], sem_i)

    def zrow(i, carry):
        for j in range(8):
            rows_a[i, pl.ds(j * 16, 16)] = jnp.zeros((16,), jnp.float32)
        return carry
    lax.fori_loop(0, CH, zrow, 0)
    base = sid * STRIPE
    for t in range(4):
        pltpu.sync_copy(rows_a.at[pl.ds(0, CH)], acc.at[pl.ds(base + t * CH, CH)])
    pltpu.sync_copy(rows_a.at[pl.ds(0, STRIPE - 4 * CH)],
                    acc.at[pl.ds(base + 4 * CH, STRIPE - 4 * CH)])

    @pl.when(sid == NS - 1)
    def _():
        pltpu.sync_copy(rows_a.at[pl.ds(0, N - NS * STRIPE)],
                        acc.at[pl.ds(NS * STRIPE, N - NS * STRIPE)])
    plsc.subcore_barrier()

    def group(g, carry):
        slot = g & 1
        pltpu.make_async_copy(src_hbm.at[g0], srcg.at[0], sem_i).wait()
        pltpu.make_async_copy(src_hbm.at[g0], dstg.at[0], sem_i).wait()

        @pl.when(g + 1 < GPT)
        def _():
            pltpu.async_copy(src_hbm.at[g0 + g + 1], srcg.at[slot ^ 1], sem_i)
            pltpu.async_copy(dst_hbm.at[g0 + g + 1], dstg.at[slot ^ 1], sem_i)

        for j in range(8):
            pltpu.async_copy(x_hbm.at[srcg.at[slot, j]], rows_a, sem_a).wait()
            pltpu.sync_copy(rows_a, acc.at[dstg.at[slot, j]], add=True)
        return carry
    lax.fori_loop(0, GPT, group, 0)

    plsc.subcore_barrier()
    pltpu.sync_copy(acc.at[pl.ds(base, STRIPE)],
                    out_hbm.at[cid, pl.ds(base, STRIPE)])

    @pl.when(sid == NS - 1)
    def _():
        pltpu.sync_copy(acc.at[pl.ds(NS * STRIPE, N - NS * STRIPE)],
                        out_hbm.at[cid, pl.ds(NS * STRIPE, N - NS * STRIPE)])


_sc_scatter = functools.partial(
    pl.kernel,
    out_type=jax.ShapeDtypeStruct((NC, N, D), jnp.float32),
    mesh=plsc.VectorSubcoreMesh(core_axis_name="c", subcore_axis_name="s"),
    scratch_types=[
        pltpu.VMEM((2, 8, CH), jnp.int32),
        pltpu.VMEM((2, 8, CH), jnp.int32),
        pltpu.VMEM((CH, D), jnp.float32),
        pltpu.VMEM((CH, D), jnp.float32),
        pltpu.VMEM_SHARED((N, D), jnp.float32),
        pltpu.SemaphoreType.DMA,
        pltpu.SemaphoreType.DMA,
        pltpu.SemaphoreType.DMA,
    ],
)(_sc_scatter_body)


def _dense_body(aggp_ref, xm_ref, wr_ref, wroot_ref, b_ref, p_ref, h_ref, s_ref):
    aggp = aggp_ref[...]
    acc = aggp[0] + aggp[1]
    pre = (jnp.dot(acc, wr_ref[...], preferred_element_type=jnp.float32)
           + jnp.dot(xm_ref[...], wroot_ref[...], preferred_element_type=jnp.float32)
           + b_ref[...])
    h = jnp.maximum(pre, 0.0)
    p = p_ref[...]
    nrm = jnp.sqrt(jnp.sum(p * p))
    s = jnp.tanh(jnp.dot(h, p, preferred_element_type=jnp.float32) / nrm)
    h_ref[...] = h
    s_ref[...] = s


def _dense(parts, xm, wr, wroot, bb, p):
    return pl.pallas_call(
        _dense_body,
        grid=(GA,),
        in_specs=[
            pl.BlockSpec((NC, RB, D), lambda i: (0, i, 0)),
            pl.BlockSpec((RB, D), lambda i: (i, 0)),
            pl.BlockSpec((D, D), lambda i: (0, 0)),
            pl.BlockSpec((D, D), lambda i: (0, 0)),
            pl.BlockSpec((1, D), lambda i: (0, 0)),
            pl.BlockSpec((D, 1), lambda i: (0, 0)),
        ],
        out_specs=[pl.BlockSpec((RB, D), lambda i: (i, 0)),
                   pl.BlockSpec((RB, 1), lambda i: (i, 0))],
        out_shape=[jax.ShapeDtypeStruct((N, D), jnp.float32),
                   jax.ShapeDtypeStruct((N, 1), jnp.float32)],
    )(parts, xm, wr, wroot, bb, p)


def _select_body(k, nprev, score_ref, mask_ref, *refs):
    prev_refs = refs[:nprev]
    selw_ref, nmask_ref, skey_ref = refs[nprev:]
    s = score_ref[...]
    m = mask_ref[...]
    ibits = lax.bitcast_convert_type(s, jnp.int32)
    skey = jnp.where(ibits < 0, ibits ^ jnp.int32(0x7FFFFFFF), ibits)
    r = lax.broadcasted_iota(jnp.int32, (SROWS, 128), 0)
    c = lax.broadcasted_iota(jnp.int32, (SROWS, 128), 1)
    gidx = r * 128 + c
    valid = (m > 0) & (gidx < N)

    eq = valid
    need = jnp.int32(k)
    sel = jnp.zeros_like(valid)
    keys = [skey] + [pr[...] for pr in prev_refs]
    for key_full in keys:
        key = jnp.where(eq, key_full, jnp.int32(IMIN))

        def tbit(i, pu):
            bb = 31 - i
            trial = pu | (jnp.int32(1) << bb)
            thr = trial ^ jnp.int32(IMIN)
            cnt = jnp.sum((key >= thr).astype(jnp.int32))
            return jnp.where(cnt >= need, trial, pu)
        pu = lax.fori_loop(0, 32, tbit, jnp.int32(0))
        t = pu ^ jnp.int32(IMIN)
        gt = eq & (key > t)
        sel = sel | gt
        need = need - jnp.sum(gt.astype(jnp.int32))
        eq = eq & (key == t)

    def jbit(i, jj):
        bb = 13 - i
        trial = jj | (jnp.int32(1) << bb)
        g = jnp.sum((eq & (gidx < trial)).astype(jnp.int32))
        return jnp.where(g < need, trial, jj)
    jmax = lax.fori_loop(0, 14, jbit, jnp.int32(0))
    sel = sel | (eq & (gidx <= jmax) & (need > 0))

    nm = sel.astype(jnp.float32)
    nmask_ref[...] = nm
    selw_ref[...] = s * nm
    skey_ref[...] = skey


def _select(k, score2d, mask2d, prev_skeys):
    nprev = len(prev_skeys)
    return pl.pallas_call(
        functools.partial(_select_body, k, nprev),
        out_shape=[jax.ShapeDtypeStruct((SROWS, 128), jnp.float32),
                   jax.ShapeDtypeStruct((SROWS, 128), jnp.float32),
                   jax.ShapeDtypeStruct((SROWS, 128), jnp.int32)],
    )(score2d, mask2d, *prev_skeys)


def _finalize_body(h_ref, selw_ref, nm_ref, y_ref, rmax_ref, rsum_ref):
    i = pl.program_id(0)
    h = h_ref[...]
    w = selw_ref[...]
    m = nm_ref[...]
    y = h * w
    y_ref[...] = y
    masked = jnp.where(m > 0, y, -jnp.inf)
    bmax = jnp.max(masked, axis=0, keepdims=True)
    bsum = jnp.sum(y, axis=0, keepdims=True)

    @pl.when(i == 0)
    def _():
        rmax_ref[...] = bmax
        rsum_ref[...] = bsum

    @pl.when(i != 0)
    def _():
        rmax_ref[...] = jnp.maximum(rmax_ref[...], bmax)
        rsum_ref[...] = rsum_ref[...] + bsum


def _finalize(h, selw, nm):
    return pl.pallas_call(
        _finalize_body,
        grid=(GA,),
        in_specs=[pl.BlockSpec((RB, D), lambda i: (i, 0)),
                  pl.BlockSpec((RB, 1), lambda i: (i, 0)),
                  pl.BlockSpec((RB, 1), lambda i: (i, 0))],
        out_specs=[pl.BlockSpec((RB, D), lambda i: (i, 0)),
                   pl.BlockSpec((1, D), lambda i: (0, 0)),
                   pl.BlockSpec((1, D), lambda i: (0, 0))],
        out_shape=[jax.ShapeDtypeStruct((N, D), jnp.float32),
                   jax.ShapeDtypeStruct((1, D), jnp.float32),
                   jax.ShapeDtypeStruct((1, D), jnp.float32)],
    )(h, selw, nm)


def _head_body(mx1, sm1, mx2, sm2, mx3, sm3, wa, wb, b1, w2, b2, w3, b3, out):
    zmax = mx1[...] + mx2[...] + mx3[...]
    zmean = sm1[...] / KS[0] + sm2[...] / KS[1] + sm3[...] / KS[2]
    a = jnp.maximum(jnp.dot(zmax, wa[...], preferred_element_type=jnp.float32)
                    + jnp.dot(zmean, wb[...], preferred_element_type=jnp.float32)
                    + b1[...], 0.0)
    a = jnp.maximum(jnp.dot(a, w2[...], preferred_element_type=jnp.float32)
                    + b2[...], 0.0)
    lg = jnp.dot(a, w3[...], preferred_element_type=jnp.float32) + b3[...]
    mx = jnp.max(lg, axis=1, keepdims=True)
    e = jnp.exp(lg - mx)
    out[...] = lg - mx - jnp.log(jnp.sum(e, axis=1, keepdims=True))


def _head(reads, L1W, L1b, L2W, L2b, L3W, L3b):
    args = []
    for rmax, rsum in reads:
        args += [rmax, rsum]
    args += [L1W[:D], L1W[D:], L1b.reshape(1, -1), L2W, L2b.reshape(1, -1),
             L3W, L3b.reshape(1, -1)]
    return pl.pallas_call(
        _head_body,
        out_shape=jax.ShapeDtypeStruct((1, 16), jnp.float32),
    )(*args)


def kernel(x, edge_index, batch, W1r, b1, W1root, p1, W2r, b2, W2root, p2,
           W3r, b3, W3root, p3, L1W, L1b, L2W, L2b, L3W, L3b):
    src = jnp.concatenate([edge_index[0], jnp.full((EPAD,), N, jnp.int32)]).reshape(NW * GPT, 8, CH)
    dst = jnp.concatenate([edge_index[1], jnp.zeros((EPAD,), jnp.int32)]).reshape(NW * GPT, 8, CH)
    Ws = ((W1r, b1, W1root, p1), (W2r, b2, W2root, p2), (W3r, b3, W3root, p3))

    xm = x
    mask2d = jnp.ones((SROWS, 128), jnp.float32)
    skeys = []
    reads = []
    for l in range(3):
        Wr, bb, Wroot, p = Ws[l]
        xmp = jnp.concatenate([xm, jnp.zeros((XP - N, D), jnp.float32)])
        parts = _sc_scatter(xmp, src, dst)
        h, score = _dense(parts, xm, Wr, Wroot, bb.reshape(1, D), p.reshape(D, 1))
        score2d = jnp.reshape(jnp.pad(score, ((0, NPAD - N), (0, 0))), (SROWS, 128))
        selw2d, mask2d, skey2d = _select(KS[l], score2d, mask2d, skeys)
        skeys.insert(0, skey2d)
        selw = jnp.reshape(selw2d, (NPAD, 1))[:N]
        nm = jnp.reshape(mask2d, (NPAD, 1))[:N]
        xm, rmax, rsum = _finalize(h, selw, nm)
        reads.append((rmax, rsum))

    return _head(reads, L1W, L1b, L2W, L2b, L3W, L3b)


# speedup vs baseline: 1.0934x; 1.0934x over previous
import functools

import jax
import jax.numpy as jnp
from jax import lax
from jax.experimental import pallas as pl
from jax.experimental.pallas import tpu as pltpu
from jax.experimental.pallas import tpu_sc as plsc

N = 10000
E = 320000
D = 128
KS = (5000, 2500, 1250)

NC = 2
NS = 16
NW = NC * NS
CH = 128
GPT = 10
CPT = GPT * 8
EPAD = NW * CPT * CH - E
XP = N + 16
STRIPE = 624

NPAD = 10240
SROWS = NPAD // 128
RB = 2000
GA = N // RB
IMIN = -2147483648


def _sc_scatter_body(x_hbm, src_hbm, dst_hbm, out_hbm,
                     srcg, dstg, rows_a, rows_b, acc, sem_i, sem_a, sem_b):
    cid = lax.axis_index("c")
    sid = lax.axis_index("s")
    wid = sid * NC + cid
    g0 = wid * GPT

    pltpu.async_copy(src_hbm.at[g0], srcg.at[0], sem_i)
    pltpu.async_copy(dst_hbm.at[g0], dstg.at[0], sem_i)

    def zrow(i, carry):
        for j in range(8):
            rows_a[i, pl.ds(j * 16, 16)] = jnp.zeros((16,), jnp.float32)
        return carry
    lax.fori_loop(0, CH, zrow, 0)
    base = sid * STRIPE
    for t in range(4):
        pltpu.sync_copy(rows_a.at[pl.ds(0, CH)], acc.at[pl.ds(base + t * CH, CH)])
    pltpu.sync_copy(rows_a.at[pl.ds(0, STRIPE - 4 * CH)],
                    acc.at[pl.ds(base + 4 * CH, STRIPE - 4 * CH)])

    @pl.when(sid == NS - 1)
    def _():
        pltpu.sync_copy(rows_a.at[pl.ds(0, N - NS * STRIPE)],
                        acc.at[pl.ds(NS * STRIPE, N - NS * STRIPE)])
    plsc.subcore_barrier()

    def group(g, carry):
        slot = g & 1
        pltpu.make_async_copy(src_hbm.at[g0], srcg.at[0], sem_i).wait()
        pltpu.make_async_copy(src_hbm.at[g0], dstg.at[0], sem_i).wait()

        @pl.when(g + 1 < GPT)
        def _():
            pltpu.async_copy(src_hbm.at[g0 + g + 1], srcg.at[slot ^ 1], sem_i)
            pltpu.async_copy(dst_hbm.at[g0 + g + 1], dstg.at[slot ^ 1], sem_i)

        pltpu.async_copy(x_hbm.at[srcg.at[slot, 0]], rows_a, sem_a)
        for pair in range(4):
            ja = 2 * pair
            jb = 2 * pair + 1
            pltpu.async_copy(x_hbm.at[srcg.at[slot, jb]], rows_b, sem_b)
            pltpu.make_async_copy(x_hbm.at[srcg.at[slot, ja]], rows_a, sem_a).wait()
            pltpu.sync_copy(rows_a, acc.at[dstg.at[slot, ja]], add=True)
            if pair < 3:
                pltpu.async_copy(x_hbm.at[srcg.at[slot, ja + 2]], rows_a, sem_a)
            pltpu.make_async_copy(x_hbm.at[srcg.at[slot, jb]], rows_b, sem_b).wait()
            pltpu.sync_copy(rows_b, acc.at[dstg.at[slot, jb]], add=True)
        return carry
    lax.fori_loop(0, GPT, group, 0)

    plsc.subcore_barrier()
    pltpu.sync_copy(acc.at[pl.ds(base, STRIPE)],
                    out_hbm.at[cid, pl.ds(base, STRIPE)])

    @pl.when(sid == NS - 1)
    def _():
        pltpu.sync_copy(acc.at[pl.ds(NS * STRIPE, N - NS * STRIPE)],
                        out_hbm.at[cid, pl.ds(NS * STRIPE, N - NS * STRIPE)])


_sc_scatter = functools.partial(
    pl.kernel,
    out_type=jax.ShapeDtypeStruct((NC, N, D), jnp.float32),
    mesh=plsc.VectorSubcoreMesh(core_axis_name="c", subcore_axis_name="s"),
    scratch_types=[
        pltpu.VMEM((2, 8, CH), jnp.int32),
        pltpu.VMEM((2, 8, CH), jnp.int32),
        pltpu.VMEM((CH, D), jnp.float32),
        pltpu.VMEM((CH, D), jnp.float32),
        pltpu.VMEM_SHARED((N, D), jnp.float32),
        pltpu.SemaphoreType.DMA,
        pltpu.SemaphoreType.DMA,
        pltpu.SemaphoreType.DMA,
    ],
)(_sc_scatter_body)


def _dense_body(aggp_ref, xm_ref, wr_ref, wroot_ref, b_ref, p_ref, h_ref, s_ref):
    aggp = aggp_ref[...]
    acc = aggp[0] + aggp[1]
    pre = (jnp.dot(acc, wr_ref[...], preferred_element_type=jnp.float32)
           + jnp.dot(xm_ref[...], wroot_ref[...], preferred_element_type=jnp.float32)
           + b_ref[...])
    h = jnp.maximum(pre, 0.0)
    p = p_ref[...]
    nrm = jnp.sqrt(jnp.sum(p * p))
    s = jnp.tanh(jnp.dot(h, p, preferred_element_type=jnp.float32) / nrm)
    h_ref[...] = h
    s_ref[...] = s


def _dense(parts, xm, wr, wroot, bb, p):
    return pl.pallas_call(
        _dense_body,
        grid=(GA,),
        in_specs=[
            pl.BlockSpec((NC, RB, D), lambda i: (0, i, 0)),
            pl.BlockSpec((RB, D), lambda i: (i, 0)),
            pl.BlockSpec((D, D), lambda i: (0, 0)),
            pl.BlockSpec((D, D), lambda i: (0, 0)),
            pl.BlockSpec((1, D), lambda i: (0, 0)),
            pl.BlockSpec((D, 1), lambda i: (0, 0)),
        ],
        out_specs=[pl.BlockSpec((RB, D), lambda i: (i, 0)),
                   pl.BlockSpec((RB, 1), lambda i: (i, 0))],
        out_shape=[jax.ShapeDtypeStruct((N, D), jnp.float32),
                   jax.ShapeDtypeStruct((N, 1), jnp.float32)],
    )(parts, xm, wr, wroot, bb, p)


def _select_body(k, nprev, score_ref, mask_ref, *refs):
    prev_refs = refs[:nprev]
    selw_ref, nmask_ref, skey_ref = refs[nprev:]
    s = score_ref[...]
    m = mask_ref[...]
    ibits = lax.bitcast_convert_type(s, jnp.int32)
    skey = jnp.where(ibits < 0, ibits ^ jnp.int32(0x7FFFFFFF), ibits)
    r = lax.broadcasted_iota(jnp.int32, (SROWS, 128), 0)
    c = lax.broadcasted_iota(jnp.int32, (SROWS, 128), 1)
    gidx = r * 128 + c
    valid = (m > 0) & (gidx < N)

    eq = valid
    need = jnp.int32(k)
    sel = jnp.zeros_like(valid)
    keys = [skey] + [pr[...] for pr in prev_refs]
    for key_full in keys:
        key = jnp.where(eq, key_full, jnp.int32(IMIN))

        def tbit(i, pu):
            bb = 31 - i
            trial = pu | (jnp.int32(1) << bb)
            thr = trial ^ jnp.int32(IMIN)
            cnt = jnp.sum((key >= thr).astype(jnp.int32))
            return jnp.where(cnt >= need, trial, pu)
        pu = lax.fori_loop(0, 32, tbit, jnp.int32(0))
        t = pu ^ jnp.int32(IMIN)
        gt = eq & (key > t)
        sel = sel | gt
        need = need - jnp.sum(gt.astype(jnp.int32))
        eq = eq & (key == t)

    def jbit(i, jj):
        bb = 13 - i
        trial = jj | (jnp.int32(1) << bb)
        g = jnp.sum((eq & (gidx < trial)).astype(jnp.int32))
        return jnp.where(g < need, trial, jj)
    jmax = lax.fori_loop(0, 14, jbit, jnp.int32(0))
    sel = sel | (eq & (gidx <= jmax) & (need > 0))

    nm = sel.astype(jnp.float32)
    nmask_ref[...] = nm
    selw_ref[...] = s * nm
    skey_ref[...] = skey


def _select(k, score2d, mask2d, prev_skeys):
    nprev = len(prev_skeys)
    return pl.pallas_call(
        functools.partial(_select_body, k, nprev),
        out_shape=[jax.ShapeDtypeStruct((SROWS, 128), jnp.float32),
                   jax.ShapeDtypeStruct((SROWS, 128), jnp.float32),
                   jax.ShapeDtypeStruct((SROWS, 128), jnp.int32)],
    )(score2d, mask2d, *prev_skeys)


def _finalize_body(h_ref, selw_ref, nm_ref, y_ref, rmax_ref, rsum_ref):
    i = pl.program_id(0)
    h = h_ref[...]
    w = selw_ref[...]
    m = nm_ref[...]
    y = h * w
    y_ref[...] = y
    masked = jnp.where(m > 0, y, -jnp.inf)
    bmax = jnp.max(masked, axis=0, keepdims=True)
    bsum = jnp.sum(y, axis=0, keepdims=True)

    @pl.when(i == 0)
    def _():
        rmax_ref[...] = bmax
        rsum_ref[...] = bsum

    @pl.when(i != 0)
    def _():
        rmax_ref[...] = jnp.maximum(rmax_ref[...], bmax)
        rsum_ref[...] = rsum_ref[...] + bsum


def _finalize(h, selw, nm):
    return pl.pallas_call(
        _finalize_body,
        grid=(GA,),
        in_specs=[pl.BlockSpec((RB, D), lambda i: (i, 0)),
                  pl.BlockSpec((RB, 1), lambda i: (i, 0)),
                  pl.BlockSpec((RB, 1), lambda i: (i, 0))],
        out_specs=[pl.BlockSpec((RB, D), lambda i: (i, 0)),
                   pl.BlockSpec((1, D), lambda i: (0, 0)),
                   pl.BlockSpec((1, D), lambda i: (0, 0))],
        out_shape=[jax.ShapeDtypeStruct((N, D), jnp.float32),
                   jax.ShapeDtypeStruct((1, D), jnp.float32),
                   jax.ShapeDtypeStruct((1, D), jnp.float32)],
    )(h, selw, nm)


def _head_body(mx1, sm1, mx2, sm2, mx3, sm3, wa, wb, b1, w2, b2, w3, b3, out):
    zmax = mx1[...] + mx2[...] + mx3[...]
    zmean = sm1[...] / KS[0] + sm2[...] / KS[1] + sm3[...] / KS[2]
    a = jnp.maximum(jnp.dot(zmax, wa[...], preferred_element_type=jnp.float32)
                    + jnp.dot(zmean, wb[...], preferred_element_type=jnp.float32)
                    + b1[...], 0.0)
    a = jnp.maximum(jnp.dot(a, w2[...], preferred_element_type=jnp.float32)
                    + b2[...], 0.0)
    lg = jnp.dot(a, w3[...], preferred_element_type=jnp.float32) + b3[...]
    mx = jnp.max(lg, axis=1, keepdims=True)
    e = jnp.exp(lg - mx)
    out[...] = lg - mx - jnp.log(jnp.sum(e, axis=1, keepdims=True))


def _head(reads, L1W, L1b, L2W, L2b, L3W, L3b):
    args = []
    for rmax, rsum in reads:
        args += [rmax, rsum]
    args += [L1W[:D], L1W[D:], L1b.reshape(1, -1), L2W, L2b.reshape(1, -1),
             L3W, L3b.reshape(1, -1)]
    return pl.pallas_call(
        _head_body,
        out_shape=jax.ShapeDtypeStruct((1, 16), jnp.float32),
    )(*args)


def kernel(x, edge_index, batch, W1r, b1, W1root, p1, W2r, b2, W2root, p2,
           W3r, b3, W3root, p3, L1W, L1b, L2W, L2b, L3W, L3b):
    src = jnp.concatenate([edge_index[0], jnp.full((EPAD,), N, jnp.int32)]).reshape(NW * GPT, 8, CH)
    dst = jnp.concatenate([edge_index[1], jnp.arange(EPAD, dtype=jnp.int32) % N]).reshape(NW * GPT, 8, CH)
    Ws = ((W1r, b1, W1root, p1), (W2r, b2, W2root, p2), (W3r, b3, W3root, p3))

    xm = x
    mask2d = jnp.ones((SROWS, 128), jnp.float32)
    skeys = []
    reads = []
    for l in range(3):
        Wr, bb, Wroot, p = Ws[l]
        xmp = jnp.concatenate([xm, jnp.zeros((XP - N, D), jnp.float32)])
        parts = _sc_scatter(xmp, src, dst)
        h, score = _dense(parts, xm, Wr, Wroot, bb.reshape(1, D), p.reshape(D, 1))
        score2d = jnp.reshape(jnp.pad(score, ((0, NPAD - N), (0, 0))), (SROWS, 128))
        selw2d, mask2d, skey2d = _select(KS[l], score2d, mask2d, skeys)
        skeys.insert(0, skey2d)
        selw = jnp.reshape(selw2d, (NPAD, 1))[:N]
        nm = jnp.reshape(mask2d, (NPAD, 1))[:N]
        xm, rmax, rsum = _finalize(h, selw, nm)
        reads.append((rmax, rsum))

    return _head(reads, L1W, L1b, L2W, L2b, L3W, L3b)


# speedup vs baseline: 2.6538x; 2.4271x over previous
import functools

import jax
import jax.numpy as jnp
from jax import lax
from jax.experimental import pallas as pl
from jax.experimental.pallas import tpu as pltpu
from jax.experimental.pallas import tpu_sc as plsc

N = 10000
E = 320000
D = 128
KS = (5000, 2500, 1250)

NC = 2
NS = 16
NW = NC * NS
CH = 128
EPW = E // NW
NFULL = EPW // CH
REM = EPW - NFULL * CH
STRIPE = 624

NPAD = 10240
SROWS = NPAD // 128
RB = 2000
GA = N // RB
IMIN = -2147483648


def _sc_scatter_body(x_hbm, src_hbm, dst_hbm, out_hbm,
                     srca, dsta, srcb, dstb, rows_a, rows_b, acc, sem_a, sem_b):
    cid = lax.axis_index("c")
    sid = lax.axis_index("s")
    wid = sid * NC + cid

    def zrow(i, carry):
        for j in range(8):
            rows_a[i, pl.ds(j * 16, 16)] = jnp.zeros((16,), jnp.float32)
        return carry
    lax.fori_loop(0, CH, zrow, 0)
    base = sid * STRIPE
    for t in range(4):
        pltpu.sync_copy(rows_a.at[pl.ds(0, CH)], acc.at[pl.ds(base + t * CH, CH)])
    pltpu.sync_copy(rows_a.at[pl.ds(0, STRIPE - 4 * CH)],
                    acc.at[pl.ds(base + 4 * CH, STRIPE - 4 * CH)])

    @pl.when(sid == NS - 1)
    def _():
        pltpu.sync_copy(rows_a.at[pl.ds(0, N - NS * STRIPE)],
                        acc.at[pl.ds(NS * STRIPE, N - NS * STRIPE)])
    plsc.subcore_barrier()

    ebase = wid * EPW

    def pair(g, carry):
        ba = ebase + (2 * g) * CH
        bb = ebase + (2 * g + 1) * CH
        pltpu.sync_copy(src_hbm.at[pl.ds(ba, CH)], srca)
        pltpu.sync_copy(dst_hbm.at[pl.ds(ba, CH)], dsta)
        ga = pltpu.async_copy(x_hbm.at[srca], rows_a, sem_a)
        pltpu.sync_copy(src_hbm.at[pl.ds(bb, CH)], srcb)
        pltpu.sync_copy(dst_hbm.at[pl.ds(bb, CH)], dstb)
        gb = pltpu.async_copy(x_hbm.at[srcb], rows_b, sem_b)
        ga.wait()
        pltpu.sync_copy(rows_a, acc.at[dsta], add=True)
        gb.wait()
        pltpu.sync_copy(rows_b, acc.at[dstb], add=True)
        return carry
    lax.fori_loop(0, NFULL // 2, pair, 0)

    b = ebase + NFULL * CH
    pltpu.sync_copy(src_hbm.at[pl.ds(b, REM)], srca.at[pl.ds(0, REM)])
    pltpu.sync_copy(dst_hbm.at[pl.ds(b, REM)], dsta.at[pl.ds(0, REM)])
    pltpu.async_copy(x_hbm.at[srca.at[pl.ds(0, REM)]], rows_a.at[pl.ds(0, REM)], sem_a).wait()
    pltpu.sync_copy(rows_a.at[pl.ds(0, REM)], acc.at[dsta.at[pl.ds(0, REM)]], add=True)

    plsc.subcore_barrier()
    pltpu.sync_copy(acc.at[pl.ds(base, STRIPE)],
                    out_hbm.at[cid, pl.ds(base, STRIPE)])

    @pl.when(sid == NS - 1)
    def _():
        pltpu.sync_copy(acc.at[pl.ds(NS * STRIPE, N - NS * STRIPE)],
                        out_hbm.at[cid, pl.ds(NS * STRIPE, N - NS * STRIPE)])


_sc_scatter = functools.partial(
    pl.kernel,
    out_type=jax.ShapeDtypeStruct((NC, N, D), jnp.float32),
    mesh=plsc.VectorSubcoreMesh(core_axis_name="c", subcore_axis_name="s"),
    scratch_types=[
        pltpu.VMEM((CH,), jnp.int32),
        pltpu.VMEM((CH,), jnp.int32),
        pltpu.VMEM((CH,), jnp.int32),
        pltpu.VMEM((CH,), jnp.int32),
        pltpu.VMEM((CH, D), jnp.float32),
        pltpu.VMEM((CH, D), jnp.float32),
        pltpu.VMEM_SHARED((N, D), jnp.float32),
        pltpu.SemaphoreType.DMA,
        pltpu.SemaphoreType.DMA,
    ],
)(_sc_scatter_body)


def _dense_body(aggp_ref, xm_ref, wr_ref, wroot_ref, b_ref, p_ref, h_ref, s_ref):
    aggp = aggp_ref[...]
    acc = aggp[0] + aggp[1]
    pre = (jnp.dot(acc, wr_ref[...], preferred_element_type=jnp.float32)
           + jnp.dot(xm_ref[...], wroot_ref[...], preferred_element_type=jnp.float32)
           + b_ref[...])
    h = jnp.maximum(pre, 0.0)
    p = p_ref[...]
    nrm = jnp.sqrt(jnp.sum(p * p))
    s = jnp.tanh(jnp.dot(h, p, preferred_element_type=jnp.float32) / nrm)
    h_ref[...] = h
    s_ref[...] = s


def _dense(parts, xm, wr, wroot, bb, p):
    return pl.pallas_call(
        _dense_body,
        grid=(GA,),
        in_specs=[
            pl.BlockSpec((NC, RB, D), lambda i: (0, i, 0)),
            pl.BlockSpec((RB, D), lambda i: (i, 0)),
            pl.BlockSpec((D, D), lambda i: (0, 0)),
            pl.BlockSpec((D, D), lambda i: (0, 0)),
            pl.BlockSpec((1, D), lambda i: (0, 0)),
            pl.BlockSpec((D, 1), lambda i: (0, 0)),
        ],
        out_specs=[pl.BlockSpec((RB, D), lambda i: (i, 0)),
                   pl.BlockSpec((RB, 1), lambda i: (i, 0))],
        out_shape=[jax.ShapeDtypeStruct((N, D), jnp.float32),
                   jax.ShapeDtypeStruct((N, 1), jnp.float32)],
    )(parts, xm, wr, wroot, bb, p)


def _select_body(k, nprev, score_ref, mask_ref, *refs):
    prev_refs = refs[:nprev]
    selw_ref, nmask_ref, skey_ref = refs[nprev:]
    s = score_ref[...]
    m = mask_ref[...]
    ibits = lax.bitcast_convert_type(s, jnp.int32)
    skey = jnp.where(ibits < 0, ibits ^ jnp.int32(0x7FFFFFFF), ibits)
    r = lax.broadcasted_iota(jnp.int32, (SROWS, 128), 0)
    c = lax.broadcasted_iota(jnp.int32, (SROWS, 128), 1)
    gidx = r * 128 + c
    valid = (m > 0) & (gidx < N)

    eq = valid
    need = jnp.int32(k)
    sel = jnp.zeros_like(valid)
    keys = [skey] + [pr[...] for pr in prev_refs]
    for key_full in keys:
        key = jnp.where(eq, key_full, jnp.int32(IMIN))

        def tbit(i, pu):
            bb = 31 - i
            trial = pu | (jnp.int32(1) << bb)
            thr = trial ^ jnp.int32(IMIN)
            cnt = jnp.sum((key >= thr).astype(jnp.int32))
            return jnp.where(cnt >= need, trial, pu)
        pu = lax.fori_loop(0, 32, tbit, jnp.int32(0))
        t = pu ^ jnp.int32(IMIN)
        gt = eq & (key > t)
        sel = sel | gt
        need = need - jnp.sum(gt.astype(jnp.int32))
        eq = eq & (key == t)

    def jbit(i, jj):
        bb = 13 - i
        trial = jj | (jnp.int32(1) << bb)
        g = jnp.sum((eq & (gidx < trial)).astype(jnp.int32))
        return jnp.where(g < need, trial, jj)
    jmax = lax.fori_loop(0, 14, jbit, jnp.int32(0))
    sel = sel | (eq & (gidx <= jmax) & (need > 0))

    nm = sel.astype(jnp.float32)
    nmask_ref[...] = nm
    selw_ref[...] = s * nm
    skey_ref[...] = skey


def _select(k, score2d, mask2d, prev_skeys):
    nprev = len(prev_skeys)
    return pl.pallas_call(
        functools.partial(_select_body, k, nprev),
        out_shape=[jax.ShapeDtypeStruct((SROWS, 128), jnp.float32),
                   jax.ShapeDtypeStruct((SROWS, 128), jnp.float32),
                   jax.ShapeDtypeStruct((SROWS, 128), jnp.int32)],
    )(score2d, mask2d, *prev_skeys)


def _finalize_body(h_ref, selw_ref, nm_ref, y_ref, rmax_ref, rsum_ref):
    i = pl.program_id(0)
    h = h_ref[...]
    w = selw_ref[...]
    m = nm_ref[...]
    y = h * w
    y_ref[...] = y
    masked = jnp.where(m > 0, y, -jnp.inf)
    bmax = jnp.max(masked, axis=0, keepdims=True)
    bsum = jnp.sum(y, axis=0, keepdims=True)

    @pl.when(i == 0)
    def _():
        rmax_ref[...] = bmax
        rsum_ref[...] = bsum

    @pl.when(i != 0)
    def _():
        rmax_ref[...] = jnp.maximum(rmax_ref[...], bmax)
        rsum_ref[...] = rsum_ref[...] + bsum


def _finalize(h, selw, nm):
    return pl.pallas_call(
        _finalize_body,
        grid=(GA,),
        in_specs=[pl.BlockSpec((RB, D), lambda i: (i, 0)),
                  pl.BlockSpec((RB, 1), lambda i: (i, 0)),
                  pl.BlockSpec((RB, 1), lambda i: (i, 0))],
        out_specs=[pl.BlockSpec((RB, D), lambda i: (i, 0)),
                   pl.BlockSpec((1, D), lambda i: (0, 0)),
                   pl.BlockSpec((1, D), lambda i: (0, 0))],
        out_shape=[jax.ShapeDtypeStruct((N, D), jnp.float32),
                   jax.ShapeDtypeStruct((1, D), jnp.float32),
                   jax.ShapeDtypeStruct((1, D), jnp.float32)],
    )(h, selw, nm)


def _head_body(mx1, sm1, mx2, sm2, mx3, sm3, wa, wb, b1, w2, b2, w3, b3, out):
    zmax = mx1[...] + mx2[...] + mx3[...]
    zmean = sm1[...] / KS[0] + sm2[...] / KS[1] + sm3[...] / KS[2]
    a = jnp.maximum(jnp.dot(zmax, wa[...], preferred_element_type=jnp.float32)
                    + jnp.dot(zmean, wb[...], preferred_element_type=jnp.float32)
                    + b1[...], 0.0)
    a = jnp.maximum(jnp.dot(a, w2[...], preferred_element_type=jnp.float32)
                    + b2[...], 0.0)
    lg = jnp.dot(a, w3[...], preferred_element_type=jnp.float32) + b3[...]
    mx = jnp.max(lg, axis=1, keepdims=True)
    e = jnp.exp(lg - mx)
    out[...] = lg - mx - jnp.log(jnp.sum(e, axis=1, keepdims=True))


def _head(reads, L1W, L1b, L2W, L2b, L3W, L3b):
    args = []
    for rmax, rsum in reads:
        args += [rmax, rsum]
    args += [L1W[:D], L1W[D:], L1b.reshape(1, -1), L2W, L2b.reshape(1, -1),
             L3W, L3b.reshape(1, -1)]
    return pl.pallas_call(
        _head_body,
        out_shape=jax.ShapeDtypeStruct((1, 16), jnp.float32),
    )(*args)


def kernel(x, edge_index, batch, W1r, b1, W1root, p1, W2r, b2, W2root, p2,
           W3r, b3, W3root, p3, L1W, L1b, L2W, L2b, L3W, L3b):
    src = edge_index[0]
    dst = edge_index[1]
    Ws = ((W1r, b1, W1root, p1), (W2r, b2, W2root, p2), (W3r, b3, W3root, p3))

    xm = x
    mask2d = jnp.ones((SROWS, 128), jnp.float32)
    skeys = []
    reads = []
    for l in range(3):
        Wr, bb, Wroot, p = Ws[l]
        parts = _sc_scatter(xm, src, dst)
        h, score = _dense(parts, xm, Wr, Wroot, bb.reshape(1, D), p.reshape(D, 1))
        score2d = jnp.reshape(jnp.pad(score, ((0, NPAD - N), (0, 0))), (SROWS, 128))
        selw2d, mask2d, skey2d = _select(KS[l], score2d, mask2d, skeys)
        skeys.insert(0, skey2d)
        selw = jnp.reshape(selw2d, (NPAD, 1))[:N]
        nm = jnp.reshape(mask2d, (NPAD, 1))[:N]
        xm, rmax, rsum = _finalize(h, selw, nm)
        reads.append((rmax, rsum))

    return _head(reads, L1W, L1b, L2W, L2b, L3W, L3b)


# speedup vs baseline: 2.9876x; 1.1258x over previous
import functools

import jax
import jax.numpy as jnp
from jax import lax
from jax.experimental import pallas as pl
from jax.experimental.pallas import tpu as pltpu
from jax.experimental.pallas import tpu_sc as plsc

N = 10000
E = 320000
D = 128
KS = (5000, 2500, 1250)

NC = 2
NS = 16
NW = NC * NS
CH = 128
EPW = E // NW
NFULL = EPW // CH
REM = EPW - NFULL * CH
STRIPE = 624

NPAD = 10240
SROWS = NPAD // 128
RB = 2000
GA = N // RB
IMIN = -2147483648


def _sc_scatter_body(x_hbm, src_hbm, dst_hbm, out_hbm,
                     srca, dsta, srca2, dsta2, srcb, dstb, srcb2, dstb2,
                     rows_a, rows_b, acc, sem_i, sem_a, sem_b, sem_sa, sem_sb):
    cid = lax.axis_index("c")
    sid = lax.axis_index("s")
    wid = sid * NC + cid

    def zrow(i, carry):
        for j in range(8):
            rows_a[i, pl.ds(j * 16, 16)] = jnp.zeros((16,), jnp.float32)
        return carry
    lax.fori_loop(0, CH, zrow, 0)
    base = sid * STRIPE
    for t in range(4):
        pltpu.sync_copy(rows_a.at[pl.ds(0, CH)], acc.at[pl.ds(base + t * CH, CH)])
    pltpu.sync_copy(rows_a.at[pl.ds(0, STRIPE - 4 * CH)],
                    acc.at[pl.ds(base + 4 * CH, STRIPE - 4 * CH)])

    @pl.when(sid == NS - 1)
    def _():
        pltpu.sync_copy(rows_a.at[pl.ds(0, N - NS * STRIPE)],
                        acc.at[pl.ds(NS * STRIPE, N - NS * STRIPE)])
    plsc.subcore_barrier()

    ebase = wid * EPW

    pltpu.async_copy(src_hbm.at[pl.ds(ebase, CH)], srca, sem_i)
    pltpu.async_copy(dst_hbm.at[pl.ds(ebase, CH)], dsta, sem_i)
    pltpu.async_copy(src_hbm.at[pl.ds(ebase + CH, CH)], srca2, sem_i)
    pltpu.async_copy(dst_hbm.at[pl.ds(ebase + CH, CH)], dsta2, sem_i)

    def quad(g, carry):
        c0 = ebase + 4 * g * CH

        def idx_wait(buf, off):
            pltpu.make_async_copy(src_hbm.at[pl.ds(c0 + off, CH)], buf, sem_i).wait()

        pltpu.async_copy(src_hbm.at[pl.ds(c0 + 2 * CH, CH)], srcb, sem_i)
        pltpu.async_copy(dst_hbm.at[pl.ds(c0 + 2 * CH, CH)], dstb, sem_i)
        pltpu.async_copy(src_hbm.at[pl.ds(c0 + 3 * CH, CH)], srcb2, sem_i)
        pltpu.async_copy(dst_hbm.at[pl.ds(c0 + 3 * CH, CH)], dstb2, sem_i)
        for _ in range(4):
            idx_wait(srca, 0)
        ga = pltpu.async_copy(x_hbm.at[srca], rows_a, sem_a)
        gb = pltpu.async_copy(x_hbm.at[srca2], rows_b, sem_b)
        ga.wait()
        sa = pltpu.async_copy(rows_a, acc.at[dsta], sem_sa, add=True)
        gb.wait()
        sb = pltpu.async_copy(rows_b, acc.at[dsta2], sem_sb, add=True)
        for _ in range(4):
            idx_wait(srcb, 2 * CH)
        sa.wait()
        ga = pltpu.async_copy(x_hbm.at[srcb], rows_a, sem_a)
        sb.wait()
        gb = pltpu.async_copy(x_hbm.at[srcb2], rows_b, sem_b)
        pltpu.async_copy(src_hbm.at[pl.ds(c0 + 4 * CH, CH)], srca, sem_i)
        pltpu.async_copy(dst_hbm.at[pl.ds(c0 + 4 * CH, CH)], dsta, sem_i)
        pltpu.async_copy(src_hbm.at[pl.ds(c0 + 5 * CH, CH)], srca2, sem_i)
        pltpu.async_copy(dst_hbm.at[pl.ds(c0 + 5 * CH, CH)], dsta2, sem_i)
        ga.wait()
        sa = pltpu.async_copy(rows_a, acc.at[dstb], sem_sa, add=True)
        gb.wait()
        sb = pltpu.async_copy(rows_b, acc.at[dstb2], sem_sb, add=True)
        sa.wait()
        sb.wait()
        return carry
    lax.fori_loop(0, NFULL // 4, quad, 0)

    ct = ebase + (NFULL - 2) * CH
    for _ in range(4):
        pltpu.make_async_copy(src_hbm.at[pl.ds(ct, CH)], srca, sem_i).wait()
    pltpu.async_copy(x_hbm.at[srca], rows_a, sem_a).wait()
    pltpu.sync_copy(rows_a, acc.at[dsta], add=True)
    pltpu.async_copy(x_hbm.at[srca2], rows_b, sem_b).wait()
    pltpu.sync_copy(rows_b, acc.at[dsta2], add=True)

    b = ebase + NFULL * CH
    pltpu.sync_copy(src_hbm.at[pl.ds(b, REM)], srca.at[pl.ds(0, REM)])
    pltpu.sync_copy(dst_hbm.at[pl.ds(b, REM)], dsta.at[pl.ds(0, REM)])
    pltpu.async_copy(x_hbm.at[srca.at[pl.ds(0, REM)]], rows_a.at[pl.ds(0, REM)], sem_a).wait()
    pltpu.sync_copy(rows_a.at[pl.ds(0, REM)], acc.at[dsta.at[pl.ds(0, REM)]], add=True)

    plsc.subcore_barrier()
    pltpu.sync_copy(acc.at[pl.ds(base, STRIPE)],
                    out_hbm.at[cid, pl.ds(base, STRIPE)])

    @pl.when(sid == NS - 1)
    def _():
        pltpu.sync_copy(acc.at[pl.ds(NS * STRIPE, N - NS * STRIPE)],
                        out_hbm.at[cid, pl.ds(NS * STRIPE, N - NS * STRIPE)])


_sc_scatter = functools.partial(
    pl.kernel,
    out_type=jax.ShapeDtypeStruct((NC, N, D), jnp.float32),
    mesh=plsc.VectorSubcoreMesh(core_axis_name="c", subcore_axis_name="s"),
    scratch_types=[
        pltpu.VMEM((CH,), jnp.int32),
        pltpu.VMEM((CH,), jnp.int32),
        pltpu.VMEM((CH,), jnp.int32),
        pltpu.VMEM((CH,), jnp.int32),
        pltpu.VMEM((CH,), jnp.int32),
        pltpu.VMEM((CH,), jnp.int32),
        pltpu.VMEM((CH,), jnp.int32),
        pltpu.VMEM((CH,), jnp.int32),
        pltpu.VMEM((CH, D), jnp.float32),
        pltpu.VMEM((CH, D), jnp.float32),
        pltpu.VMEM_SHARED((N, D), jnp.float32),
        pltpu.SemaphoreType.DMA,
        pltpu.SemaphoreType.DMA,
        pltpu.SemaphoreType.DMA,
        pltpu.SemaphoreType.DMA,
        pltpu.SemaphoreType.DMA,
    ],
)(_sc_scatter_body)


def _dense_body(aggp_ref, xm_ref, wr_ref, wroot_ref, b_ref, p_ref, h_ref, s_ref):
    aggp = aggp_ref[...]
    acc = aggp[0] + aggp[1]
    pre = (jnp.dot(acc, wr_ref[...], preferred_element_type=jnp.float32)
           + jnp.dot(xm_ref[...], wroot_ref[...], preferred_element_type=jnp.float32)
           + b_ref[...])
    h = jnp.maximum(pre, 0.0)
    p = p_ref[...]
    nrm = jnp.sqrt(jnp.sum(p * p))
    s = jnp.tanh(jnp.dot(h, p, preferred_element_type=jnp.float32) / nrm)
    h_ref[...] = h
    s_ref[...] = s


def _dense(parts, xm, wr, wroot, bb, p):
    return pl.pallas_call(
        _dense_body,
        grid=(GA,),
        in_specs=[
            pl.BlockSpec((NC, RB, D), lambda i: (0, i, 0)),
            pl.BlockSpec((RB, D), lambda i: (i, 0)),
            pl.BlockSpec((D, D), lambda i: (0, 0)),
            pl.BlockSpec((D, D), lambda i: (0, 0)),
            pl.BlockSpec((1, D), lambda i: (0, 0)),
            pl.BlockSpec((D, 1), lambda i: (0, 0)),
        ],
        out_specs=[pl.BlockSpec((RB, D), lambda i: (i, 0)),
                   pl.BlockSpec((RB, 1), lambda i: (i, 0))],
        out_shape=[jax.ShapeDtypeStruct((N, D), jnp.float32),
                   jax.ShapeDtypeStruct((N, 1), jnp.float32)],
    )(parts, xm, wr, wroot, bb, p)


def _select_body(k, nprev, score_ref, mask_ref, *refs):
    prev_refs = refs[:nprev]
    selw_ref, nmask_ref, skey_ref = refs[nprev:]
    s = score_ref[...]
    m = mask_ref[...]
    ibits = lax.bitcast_convert_type(s, jnp.int32)
    skey = jnp.where(ibits < 0, ibits ^ jnp.int32(0x7FFFFFFF), ibits)
    r = lax.broadcasted_iota(jnp.int32, (SROWS, 128), 0)
    c = lax.broadcasted_iota(jnp.int32, (SROWS, 128), 1)
    gidx = r * 128 + c
    valid = (m > 0) & (gidx < N)

    eq = valid
    need = jnp.int32(k)
    sel = jnp.zeros_like(valid)
    keys = [skey] + [pr[...] for pr in prev_refs]
    for key_full in keys:
        key = jnp.where(eq, key_full, jnp.int32(IMIN))

        def tbit(i, pu):
            bb = 31 - i
            trial = pu | (jnp.int32(1) << bb)
            thr = trial ^ jnp.int32(IMIN)
            cnt = jnp.sum((key >= thr).astype(jnp.int32))
            return jnp.where(cnt >= need, trial, pu)
        pu = lax.fori_loop(0, 32, tbit, jnp.int32(0))
        t = pu ^ jnp.int32(IMIN)
        gt = eq & (key > t)
        sel = sel | gt
        need = need - jnp.sum(gt.astype(jnp.int32))
        eq = eq & (key == t)

    def jbit(i, jj):
        bb = 13 - i
        trial = jj | (jnp.int32(1) << bb)
        g = jnp.sum((eq & (gidx < trial)).astype(jnp.int32))
        return jnp.where(g < need, trial, jj)
    jmax = lax.fori_loop(0, 14, jbit, jnp.int32(0))
    sel = sel | (eq & (gidx <= jmax) & (need > 0))

    nm = sel.astype(jnp.float32)
    nmask_ref[...] = nm
    selw_ref[...] = s * nm
    skey_ref[...] = skey


def _select(k, score2d, mask2d, prev_skeys):
    nprev = len(prev_skeys)
    return pl.pallas_call(
        functools.partial(_select_body, k, nprev),
        out_shape=[jax.ShapeDtypeStruct((SROWS, 128), jnp.float32),
                   jax.ShapeDtypeStruct((SROWS, 128), jnp.float32),
                   jax.ShapeDtypeStruct((SROWS, 128), jnp.int32)],
    )(score2d, mask2d, *prev_skeys)


def _finalize_body(h_ref, selw_ref, nm_ref, y_ref, rmax_ref, rsum_ref):
    i = pl.program_id(0)
    h = h_ref[...]
    w = selw_ref[...]
    m = nm_ref[...]
    y = h * w
    y_ref[...] = y
    masked = jnp.where(m > 0, y, -jnp.inf)
    bmax = jnp.max(masked, axis=0, keepdims=True)
    bsum = jnp.sum(y, axis=0, keepdims=True)

    @pl.when(i == 0)
    def _():
        rmax_ref[...] = bmax
        rsum_ref[...] = bsum

    @pl.when(i != 0)
    def _():
        rmax_ref[...] = jnp.maximum(rmax_ref[...], bmax)
        rsum_ref[...] = rsum_ref[...] + bsum


def _finalize(h, selw, nm):
    return pl.pallas_call(
        _finalize_body,
        grid=(GA,),
        in_specs=[pl.BlockSpec((RB, D), lambda i: (i, 0)),
                  pl.BlockSpec((RB, 1), lambda i: (i, 0)),
                  pl.BlockSpec((RB, 1), lambda i: (i, 0))],
        out_specs=[pl.BlockSpec((RB, D), lambda i: (i, 0)),
                   pl.BlockSpec((1, D), lambda i: (0, 0)),
                   pl.BlockSpec((1, D), lambda i: (0, 0))],
        out_shape=[jax.ShapeDtypeStruct((N, D), jnp.float32),
                   jax.ShapeDtypeStruct((1, D), jnp.float32),
                   jax.ShapeDtypeStruct((1, D), jnp.float32)],
    )(h, selw, nm)


def _head_body(mx1, sm1, mx2, sm2, mx3, sm3, wa, wb, b1, w2, b2, w3, b3, out):
    zmax = mx1[...] + mx2[...] + mx3[...]
    zmean = sm1[...] / KS[0] + sm2[...] / KS[1] + sm3[...] / KS[2]
    a = jnp.maximum(jnp.dot(zmax, wa[...], preferred_element_type=jnp.float32)
                    + jnp.dot(zmean, wb[...], preferred_element_type=jnp.float32)
                    + b1[...], 0.0)
    a = jnp.maximum(jnp.dot(a, w2[...], preferred_element_type=jnp.float32)
                    + b2[...], 0.0)
    lg = jnp.dot(a, w3[...], preferred_element_type=jnp.float32) + b3[...]
    mx = jnp.max(lg, axis=1, keepdims=True)
    e = jnp.exp(lg - mx)
    out[...] = lg - mx - jnp.log(jnp.sum(e, axis=1, keepdims=True))


def _head(reads, L1W, L1b, L2W, L2b, L3W, L3b):
    args = []
    for rmax, rsum in reads:
        args += [rmax, rsum]
    args += [L1W[:D], L1W[D:], L1b.reshape(1, -1), L2W, L2b.reshape(1, -1),
             L3W, L3b.reshape(1, -1)]
    return pl.pallas_call(
        _head_body,
        out_shape=jax.ShapeDtypeStruct((1, 16), jnp.float32),
    )(*args)


def kernel(x, edge_index, batch, W1r, b1, W1root, p1, W2r, b2, W2root, p2,
           W3r, b3, W3root, p3, L1W, L1b, L2W, L2b, L3W, L3b):
    src = edge_index[0]
    dst = edge_index[1]
    Ws = ((W1r, b1, W1root, p1), (W2r, b2, W2root, p2), (W3r, b3, W3root, p3))

    xm = x
    mask2d = jnp.ones((SROWS, 128), jnp.float32)
    skeys = []
    reads = []
    for l in range(3):
        Wr, bb, Wroot, p = Ws[l]
        parts = _sc_scatter(xm, src, dst)
        h, score = _dense(parts, xm, Wr, Wroot, bb.reshape(1, D), p.reshape(D, 1))
        score2d = jnp.reshape(jnp.pad(score, ((0, NPAD - N), (0, 0))), (SROWS, 128))
        selw2d, mask2d, skey2d = _select(KS[l], score2d, mask2d, skeys)
        skeys.insert(0, skey2d)
        selw = jnp.reshape(selw2d, (NPAD, 1))[:N]
        nm = jnp.reshape(mask2d, (NPAD, 1))[:N]
        xm, rmax, rsum = _finalize(h, selw, nm)
        reads.append((rmax, rsum))

    return _head(reads, L1W, L1b, L2W, L2b, L3W, L3b)


# speedup vs baseline: 2.9938x; 1.0021x over previous
import functools

import jax
import jax.numpy as jnp
from jax import lax
from jax.experimental import pallas as pl
from jax.experimental.pallas import tpu as pltpu
from jax.experimental.pallas import tpu_sc as plsc

N = 10000
E = 320000
D = 128
KS = (5000, 2500, 1250)

NC = 2
NS = 16
NW = NC * NS
CH = 128
EPW = E // NW
NFULL = EPW // CH
REM = EPW - NFULL * CH
STRIPE = 624

NPAD = 10240
SROWS = NPAD // 128
RB = 2000
GA = N // RB
IMIN = -2147483648


def _sc_scatter_body(x_hbm, src_hbm, dst_hbm, out_hbm,
                     srca, dsta, srca2, dsta2, srcb, dstb, srcb2, dstb2,
                     rows_a, rows_b, acc, sem_i, sem_a, sem_b, sem_sa, sem_sb):
    cid = lax.axis_index("c")
    sid = lax.axis_index("s")
    wid = sid * NC + cid

    def zrow(i, carry):
        for j in range(8):
            rows_a[i, pl.ds(j * 16, 16)] = jnp.zeros((16,), jnp.float32)
        return carry
    lax.fori_loop(0, CH, zrow, 0)
    base = sid * STRIPE
    for t in range(4):
        pltpu.sync_copy(rows_a.at[pl.ds(0, CH)], acc.at[pl.ds(base + t * CH, CH)])
    pltpu.sync_copy(rows_a.at[pl.ds(0, STRIPE - 4 * CH)],
                    acc.at[pl.ds(base + 4 * CH, STRIPE - 4 * CH)])

    @pl.when(sid == NS - 1)
    def _():
        pltpu.sync_copy(rows_a.at[pl.ds(0, N - NS * STRIPE)],
                        acc.at[pl.ds(NS * STRIPE, N - NS * STRIPE)])
    plsc.subcore_barrier()

    ebase = wid * EPW

    pltpu.async_copy(src_hbm.at[pl.ds(ebase, CH)], srca, sem_i)
    pltpu.async_copy(dst_hbm.at[pl.ds(ebase, CH)], dsta, sem_i)
    pltpu.async_copy(src_hbm.at[pl.ds(ebase + CH, CH)], srca2, sem_i)
    pltpu.async_copy(dst_hbm.at[pl.ds(ebase + CH, CH)], dsta2, sem_i)

    def quad(g, carry):
        c0 = ebase + 4 * g * CH

        def idx_wait(buf, off):
            pltpu.make_async_copy(src_hbm.at[pl.ds(c0 + off, CH)], buf, sem_i).wait()

        pltpu.async_copy(src_hbm.at[pl.ds(c0 + 2 * CH, CH)], srcb, sem_i)
        pltpu.async_copy(dst_hbm.at[pl.ds(c0 + 2 * CH, CH)], dstb, sem_i)
        pltpu.async_copy(src_hbm.at[pl.ds(c0 + 3 * CH, CH)], srcb2, sem_i)
        pltpu.async_copy(dst_hbm.at[pl.ds(c0 + 3 * CH, CH)], dstb2, sem_i)
        for _ in range(4):
            idx_wait(srca, 0)
        ga = pltpu.async_copy(x_hbm.at[srca], rows_a, sem_a)
        gb = pltpu.async_copy(x_hbm.at[srca2], rows_b, sem_b)
        ga.wait()
        sa = pltpu.async_copy(rows_a, acc.at[dsta], sem_sa, add=True)
        gb.wait()
        sb = pltpu.async_copy(rows_b, acc.at[dsta2], sem_sb, add=True)
        for _ in range(4):
            idx_wait(srcb, 2 * CH)
        sa.wait()
        ga = pltpu.async_copy(x_hbm.at[srcb], rows_a, sem_a)
        sb.wait()
        gb = pltpu.async_copy(x_hbm.at[srcb2], rows_b, sem_b)
        pltpu.async_copy(src_hbm.at[pl.ds(c0 + 4 * CH, CH)], srca, sem_i)
        pltpu.async_copy(dst_hbm.at[pl.ds(c0 + 4 * CH, CH)], dsta, sem_i)
        pltpu.async_copy(src_hbm.at[pl.ds(c0 + 5 * CH, CH)], srca2, sem_i)
        pltpu.async_copy(dst_hbm.at[pl.ds(c0 + 5 * CH, CH)], dsta2, sem_i)
        ga.wait()
        sa = pltpu.async_copy(rows_a, acc.at[dstb], sem_sa, add=True)
        gb.wait()
        sb = pltpu.async_copy(rows_b, acc.at[dstb2], sem_sb, add=True)
        sa.wait()
        sb.wait()
        return carry
    lax.fori_loop(0, NFULL // 4, quad, 0)

    ct = ebase + (NFULL - 2) * CH
    for _ in range(4):
        pltpu.make_async_copy(src_hbm.at[pl.ds(ct, CH)], srca, sem_i).wait()
    pltpu.async_copy(x_hbm.at[srca], rows_a, sem_a).wait()
    pltpu.sync_copy(rows_a, acc.at[dsta], add=True)
    pltpu.async_copy(x_hbm.at[srca2], rows_b, sem_b).wait()
    pltpu.sync_copy(rows_b, acc.at[dsta2], add=True)

    b = ebase + NFULL * CH
    pltpu.sync_copy(src_hbm.at[pl.ds(b, REM)], srca.at[pl.ds(0, REM)])
    pltpu.sync_copy(dst_hbm.at[pl.ds(b, REM)], dsta.at[pl.ds(0, REM)])
    pltpu.async_copy(x_hbm.at[srca.at[pl.ds(0, REM)]], rows_a.at[pl.ds(0, REM)], sem_a).wait()
    pltpu.sync_copy(rows_a.at[pl.ds(0, REM)], acc.at[dsta.at[pl.ds(0, REM)]], add=True)

    plsc.subcore_barrier()
    pltpu.sync_copy(acc.at[pl.ds(base, STRIPE)],
                    out_hbm.at[cid, pl.ds(base, STRIPE)])

    @pl.when(sid == NS - 1)
    def _():
        pltpu.sync_copy(acc.at[pl.ds(NS * STRIPE, N - NS * STRIPE)],
                        out_hbm.at[cid, pl.ds(NS * STRIPE, N - NS * STRIPE)])


_sc_scatter = functools.partial(
    pl.kernel,
    out_type=jax.ShapeDtypeStruct((NC, N, D), jnp.float32),
    mesh=plsc.VectorSubcoreMesh(core_axis_name="c", subcore_axis_name="s"),
    scratch_types=[
        pltpu.VMEM((CH,), jnp.int32),
        pltpu.VMEM((CH,), jnp.int32),
        pltpu.VMEM((CH,), jnp.int32),
        pltpu.VMEM((CH,), jnp.int32),
        pltpu.VMEM((CH,), jnp.int32),
        pltpu.VMEM((CH,), jnp.int32),
        pltpu.VMEM((CH,), jnp.int32),
        pltpu.VMEM((CH,), jnp.int32),
        pltpu.VMEM((CH, D), jnp.float32),
        pltpu.VMEM((CH, D), jnp.float32),
        pltpu.VMEM_SHARED((N, D), jnp.float32),
        pltpu.SemaphoreType.DMA,
        pltpu.SemaphoreType.DMA,
        pltpu.SemaphoreType.DMA,
        pltpu.SemaphoreType.DMA,
        pltpu.SemaphoreType.DMA,
    ],
)(_sc_scatter_body)


def _dense_body(aggp_ref, xm_ref, wr_ref, wroot_ref, b_ref, p_ref, h_ref, s_ref):
    aggp = aggp_ref[...]
    agg = aggp[0] + aggp[1]
    pre = (jnp.dot(agg, wr_ref[...], preferred_element_type=jnp.float32)
           + jnp.dot(xm_ref[...], wroot_ref[...], preferred_element_type=jnp.float32)
           + b_ref[...])
    h = jnp.maximum(pre, 0.0)
    p = p_ref[...]
    nrm = jnp.sqrt(jnp.sum(p * p))
    h_ref[...] = h
    s_ref[...] = jnp.tanh(jnp.dot(h, p, preferred_element_type=jnp.float32) / nrm)


def _dense(parts, xm, wr, wroot, bb, p):
    return pl.pallas_call(
        _dense_body,
        grid=(GA,),
        in_specs=[
            pl.BlockSpec((NC, RB, D), lambda i: (0, i, 0)),
            pl.BlockSpec((RB, D), lambda i: (i, 0)),
            pl.BlockSpec((D, D), lambda i: (0, 0)),
            pl.BlockSpec((D, D), lambda i: (0, 0)),
            pl.BlockSpec((1, D), lambda i: (0, 0)),
            pl.BlockSpec((D, 1), lambda i: (0, 0)),
        ],
        out_specs=[pl.BlockSpec((RB, D), lambda i: (i, 0)),
                   pl.BlockSpec((RB, 1), lambda i: (i, 0))],
        out_shape=[jax.ShapeDtypeStruct((N, D), jnp.float32),
                   jax.ShapeDtypeStruct((N, 1), jnp.float32)],
    )(parts, xm, wr, wroot, bb, p)


def _sortable(s):
    ibits = lax.bitcast_convert_type(s, jnp.int32)
    return jnp.where(ibits < 0, ibits ^ jnp.int32(0x7FFFFFFF), ibits)


def _select_thresholds(k, s2, m2, prev_keys2):
    skey = _sortable(s2)
    r = lax.broadcasted_iota(jnp.int32, (SROWS, 128), 0)
    c = lax.broadcasted_iota(jnp.int32, (SROWS, 128), 1)
    gidx = r * 128 + c
    valid = (m2 > 0) & (gidx < N)

    eq = valid
    need = jnp.int32(k)
    ts = []
    for key_full in [skey] + prev_keys2:
        key = jnp.where(eq, key_full, jnp.int32(IMIN))

        def tbit(i, pu):
            bb = 31 - i
            trial = pu | (jnp.int32(1) << bb)
            thr = trial ^ jnp.int32(IMIN)
            cnt = jnp.sum((key >= thr).astype(jnp.int32))
            return jnp.where(cnt >= need, trial, pu)
        pu = lax.fori_loop(0, 32, tbit, jnp.int32(0))
        t = pu ^ jnp.int32(IMIN)
        ts.append(t)
        need = need - jnp.sum((eq & (key > t)).astype(jnp.int32))
        eq = eq & (key == t)

    def jbit(i, jj):
        bb = 13 - i
        trial = jj | (jnp.int32(1) << bb)
        g = jnp.sum((eq & (gidx < trial)).astype(jnp.int32))
        return jnp.where(g < need, trial, jj)
    jmax = lax.fori_loop(0, 14, jbit, jnp.int32(0))
    return skey, ts, jmax, need


def _reconstruct(ts, jmax, needf, valid, keys, gidx):
    eq = valid
    sel = jnp.zeros_like(valid)
    for key_full, t in zip(keys, ts):
        key = jnp.where(eq, key_full, jnp.int32(IMIN))
        sel = sel | (eq & (key > t))
        eq = eq & (key == t)
    return sel | (eq & (gidx <= jmax) & (needf > 0))


def _layerB_body(k, nprev, h_ref, s_ref, s2_ref, m2_ref, mcol_ref, *refs):
    prev2 = [refs[i][...] for i in range(nprev)]
    prevcol = [refs[nprev + i][...] for i in range(nprev)]
    (y_ref, nm2_ref, nmcol_ref, skey2_ref, skeycol_ref,
     rmax_ref, rsum_ref) = refs[2 * nprev:]

    s2 = s2_ref[...]
    skey2, ts, jmax, needf = _select_thresholds(k, s2, m2_ref[...], prev2)
    sel2 = _reconstruct(
        ts, jmax, needf,
        (m2_ref[...] > 0) & ((lax.broadcasted_iota(jnp.int32, (SROWS, 128), 0) * 128
                              + lax.broadcasted_iota(jnp.int32, (SROWS, 128), 1)) < N),
        [skey2] + prev2,
        lax.broadcasted_iota(jnp.int32, (SROWS, 128), 0) * 128
        + lax.broadcasted_iota(jnp.int32, (SROWS, 128), 1))

    s = s_ref[...]
    skeycol = _sortable(s)
    gcol = lax.broadcasted_iota(jnp.int32, (N, 1), 0)
    selcol = _reconstruct(ts, jmax, needf, mcol_ref[...] > 0,
                          [skeycol] + prevcol, gcol)
    nmc = selcol.astype(jnp.float32)

    h = h_ref[...]
    y = h * (s * nmc)
    rmax = jnp.max(jnp.where(selcol, y, -jnp.inf), axis=0, keepdims=True)
    rsum = jnp.sum(y, axis=0, keepdims=True)

    y_ref[...] = y
    nm2_ref[...] = sel2.astype(jnp.float32)
    nmcol_ref[...] = nmc
    skey2_ref[...] = skey2
    skeycol_ref[...] = skeycol
    rmax_ref[...] = rmax
    rsum_ref[...] = rsum


def _layerB(k, nprev, h, s, s2, m2, mcol, prev2, prevcol):
    return pl.pallas_call(
        functools.partial(_layerB_body, k, nprev),
        out_shape=[jax.ShapeDtypeStruct((N, D), jnp.float32),
                   jax.ShapeDtypeStruct((SROWS, 128), jnp.float32),
                   jax.ShapeDtypeStruct((N, 1), jnp.float32),
                   jax.ShapeDtypeStruct((SROWS, 128), jnp.int32),
                   jax.ShapeDtypeStruct((N, 1), jnp.int32),
                   jax.ShapeDtypeStruct((1, D), jnp.float32),
                   jax.ShapeDtypeStruct((1, D), jnp.float32)],
    )(h, s, s2, m2, mcol, *prev2, *prevcol)


def _layer3B_body(k, nprev, h_ref, s_ref, s2_ref, m2_ref, mcol_ref, *refs):
    prev2 = [refs[i][...] for i in range(nprev)]
    prevcol = [refs[nprev + i][...] for i in range(nprev)]
    (mx1, sm1, mx2, sm2, wa, wb, b1, w2, b2, w3, b3, out_ref) = refs[2 * nprev:]

    s2 = s2_ref[...]
    skey2, ts, jmax, needf = _select_thresholds(k, s2, m2_ref[...], prev2)
    s = s_ref[...]
    skeycol = _sortable(s)
    gcol = lax.broadcasted_iota(jnp.int32, (N, 1), 0)
    selcol = _reconstruct(ts, jmax, needf, mcol_ref[...] > 0,
                          [skeycol] + prevcol, gcol)
    nmc = selcol.astype(jnp.float32)

    h = h_ref[...]
    y = h * (s * nmc)
    rmax = jnp.max(jnp.where(selcol, y, -jnp.inf), axis=0, keepdims=True)
    rsum = jnp.sum(y, axis=0, keepdims=True)

    zmax = mx1[...] + mx2[...] + rmax
    zmean = sm1[...] / KS[0] + sm2[...] / KS[1] + rsum / KS[2]
    a = jnp.maximum(jnp.dot(zmax, wa[...], preferred_element_type=jnp.float32)
                    + jnp.dot(zmean, wb[...], preferred_element_type=jnp.float32)
                    + b1[...], 0.0)
    a = jnp.maximum(jnp.dot(a, w2[...], preferred_element_type=jnp.float32)
                    + b2[...], 0.0)
    lg = jnp.dot(a, w3[...], preferred_element_type=jnp.float32) + b3[...]
    mx = jnp.max(lg, axis=1, keepdims=True)
    e = jnp.exp(lg - mx)
    out_ref[...] = lg - mx - jnp.log(jnp.sum(e, axis=1, keepdims=True))


def _layer3B(k, nprev, h, s, s2, m2, mcol, prev2, prevcol, reads,
             L1W, L1b, L2W, L2b, L3W, L3b):
    (mx1, sm1), (mx2, sm2) = reads
    return pl.pallas_call(
        functools.partial(_layer3B_body, k, nprev),
        out_shape=jax.ShapeDtypeStruct((1, 16), jnp.float32),
    )(h, s, s2, m2, mcol, *prev2, *prevcol,
      mx1, sm1, mx2, sm2, L1W[:D], L1W[D:], L1b.reshape(1, -1),
      L2W, L2b.reshape(1, -1), L3W, L3b.reshape(1, -1))


def kernel(x, edge_index, batch, W1r, b1, W1root, p1, W2r, b2, W2root, p2,
           W3r, b3, W3root, p3, L1W, L1b, L2W, L2b, L3W, L3b):
    src = edge_index[0]
    dst = edge_index[1]
    Ws = ((W1r, b1, W1root, p1), (W2r, b2, W2root, p2), (W3r, b3, W3root, p3))

    xm = x
    m2 = jnp.ones((SROWS, 128), jnp.float32)
    mcol = jnp.ones((N, 1), jnp.float32)
    prev2, prevcol = [], []
    reads = []
    for l in range(2):
        Wr, bb, Wroot, p = Ws[l]
        parts = _sc_scatter(xm, src, dst)
        h, s = _dense(parts, xm, Wr, Wroot, bb.reshape(1, D), p.reshape(D, 1))
        s2 = jnp.reshape(jnp.pad(s, ((0, NPAD - N), (0, 0))), (SROWS, 128))
        (xm, m2, mcol, skey2, skeycol, rmax, rsum) = _layerB(
            KS[l], len(prev2), h, s, s2, m2, mcol, prev2, prevcol)
        prev2.insert(0, skey2)
        prevcol.insert(0, skeycol)
        reads.append((rmax, rsum))

    Wr, bb, Wroot, p = Ws[2]
    parts = _sc_scatter(xm, src, dst)
    h, s = _dense(parts, xm, Wr, Wroot, bb.reshape(1, D), p.reshape(D, 1))
    s2 = jnp.reshape(jnp.pad(s, ((0, NPAD - N), (0, 0))), (SROWS, 128))
    return _layer3B(KS[2], len(prev2), h, s, s2, m2, mcol, prev2, prevcol,
                    reads, L1W, L1b, L2W, L2b, L3W, L3b)


# speedup vs baseline: 3.2520x; 1.0862x over previous
import functools

import jax
import jax.numpy as jnp
from jax import lax
from jax.experimental import pallas as pl
from jax.experimental.pallas import tpu as pltpu
from jax.experimental.pallas import tpu_sc as plsc

N = 10000
E = 320000
D = 128
KS = (5000, 2500, 1250)

NC = 2
NS = 16
NW = NC * NS
CH = 128
EPW = E // NW
NFULL = EPW // CH
REM = EPW - NFULL * CH
STRIPE = 624

NPAD = 10240
SROWS = NPAD // 128
RB = 2000
GA = N // RB
IMIN = -2147483648


def _sc_scatter_body(x_hbm, src_hbm, dst_hbm, out_hbm,
                     s0, d0, s1, d1, s2, d2, s3, d3, s4, d4, s5, d5,
                     rows_a, rows_b, rows_c, acc,
                     sem_i, sem_ga, sem_gb, sem_gc, sem_sa, sem_sb, sem_sc):
    srcs = [s0, s1, s2, s3, s4, s5]
    dsts = [d0, d1, d2, d3, d4, d5]
    cid = lax.axis_index("c")
    sid = lax.axis_index("s")
    wid = sid * NC + cid

    def zrow(i, carry):
        for j in range(8):
            rows_a[i, pl.ds(j * 16, 16)] = jnp.zeros((16,), jnp.float32)
        return carry
    lax.fori_loop(0, CH, zrow, 0)
    base = sid * STRIPE
    for t in range(4):
        pltpu.sync_copy(rows_a.at[pl.ds(0, CH)], acc.at[pl.ds(base + t * CH, CH)])
    pltpu.sync_copy(rows_a.at[pl.ds(0, STRIPE - 4 * CH)],
                    acc.at[pl.ds(base + 4 * CH, STRIPE - 4 * CH)])

    @pl.when(sid == NS - 1)
    def _():
        pltpu.sync_copy(rows_a.at[pl.ds(0, N - NS * STRIPE)],
                        acc.at[pl.ds(NS * STRIPE, N - NS * STRIPE)])
    plsc.subcore_barrier()

    ebase = wid * EPW

    for t in range(3):
        pltpu.async_copy(src_hbm.at[pl.ds(ebase + t * CH, CH)], srcs[t], sem_i)
        pltpu.async_copy(dst_hbm.at[pl.ds(ebase + t * CH, CH)], dsts[t], sem_i)

    def hexad(g, carry):
        c0 = ebase + 6 * g * CH

        def idx_wait():
            pltpu.make_async_copy(src_hbm.at[pl.ds(c0, CH)], srcs[0], sem_i).wait()

        for t in range(3, 6):
            pltpu.async_copy(src_hbm.at[pl.ds(c0 + t * CH, CH)], srcs[t], sem_i)
            pltpu.async_copy(dst_hbm.at[pl.ds(c0 + t * CH, CH)], dsts[t], sem_i)
        for _ in range(6):
            idx_wait()
        g0 = pltpu.async_copy(x_hbm.at[srcs[0]], rows_a, sem_ga)
        g1 = pltpu.async_copy(x_hbm.at[srcs[1]], rows_b, sem_gb)
        g2 = pltpu.async_copy(x_hbm.at[srcs[2]], rows_c, sem_gc)
        g0.wait()
        s0 = pltpu.async_copy(rows_a, acc.at[dsts[0]], sem_sa, add=True)
        g1.wait()
        s1 = pltpu.async_copy(rows_b, acc.at[dsts[1]], sem_sb, add=True)
        g2.wait()
        s2 = pltpu.async_copy(rows_c, acc.at[dsts[2]], sem_sc, add=True)
        for _ in range(6):
            idx_wait()
        s0.wait()
        g0 = pltpu.async_copy(x_hbm.at[srcs[3]], rows_a, sem_ga)
        s1.wait()
        g1 = pltpu.async_copy(x_hbm.at[srcs[4]], rows_b, sem_gb)
        s2.wait()
        g2 = pltpu.async_copy(x_hbm.at[srcs[5]], rows_c, sem_gc)

        @pl.when(g + 1 < NFULL // 6)
        def _():
            for t in range(3):
                pltpu.async_copy(src_hbm.at[pl.ds(c0 + (6 + t) * CH, CH)], srcs[t], sem_i)
                pltpu.async_copy(dst_hbm.at[pl.ds(c0 + (6 + t) * CH, CH)], dsts[t], sem_i)
        g0.wait()
        s0 = pltpu.async_copy(rows_a, acc.at[dsts[3]], sem_sa, add=True)
        g1.wait()
        s1 = pltpu.async_copy(rows_b, acc.at[dsts[4]], sem_sb, add=True)
        g2.wait()
        s2 = pltpu.async_copy(rows_c, acc.at[dsts[5]], sem_sc, add=True)
        s0.wait()
        s1.wait()
        s2.wait()
        return carry
    lax.fori_loop(0, NFULL // 6, hexad, 0)

    b = ebase + NFULL * CH
    pltpu.sync_copy(src_hbm.at[pl.ds(b, REM)], srcs[0].at[pl.ds(0, REM)])
    pltpu.sync_copy(dst_hbm.at[pl.ds(b, REM)], dsts[0].at[pl.ds(0, REM)])
    pltpu.async_copy(x_hbm.at[srcs[0].at[pl.ds(0, REM)]], rows_a.at[pl.ds(0, REM)], sem_ga).wait()
    pltpu.sync_copy(rows_a.at[pl.ds(0, REM)], acc.at[dsts[0].at[pl.ds(0, REM)]], add=True)

    plsc.subcore_barrier()
    pltpu.sync_copy(acc.at[pl.ds(base, STRIPE)],
                    out_hbm.at[cid, pl.ds(base, STRIPE)])

    @pl.when(sid == NS - 1)
    def _():
        pltpu.sync_copy(acc.at[pl.ds(NS * STRIPE, N - NS * STRIPE)],
                        out_hbm.at[cid, pl.ds(NS * STRIPE, N - NS * STRIPE)])


_sc_scatter = functools.partial(
    pl.kernel,
    out_type=jax.ShapeDtypeStruct((NC, N, D), jnp.float32),
    mesh=plsc.VectorSubcoreMesh(core_axis_name="c", subcore_axis_name="s"),
    scratch_types=(
        [pltpu.VMEM((CH,), jnp.int32)] * 12
        + [pltpu.VMEM((CH, D), jnp.float32)] * 3
        + [pltpu.VMEM_SHARED((N, D), jnp.float32)]
        + [pltpu.SemaphoreType.DMA] * 7
    ),
)(_sc_scatter_body)


def _dense_body(aggp_ref, xm_ref, wr_ref, wroot_ref, b_ref, p_ref, h_ref, s_ref):
    aggp = aggp_ref[...]
    agg = aggp[0] + aggp[1]
    pre = (jnp.dot(agg, wr_ref[...], preferred_element_type=jnp.float32)
           + jnp.dot(xm_ref[...], wroot_ref[...], preferred_element_type=jnp.float32)
           + b_ref[...])
    h = jnp.maximum(pre, 0.0)
    p = p_ref[...]
    nrm = jnp.sqrt(jnp.sum(p * p))
    h_ref[...] = h
    s_ref[...] = jnp.tanh(jnp.dot(h, p, preferred_element_type=jnp.float32) / nrm)


def _dense(parts, xm, wr, wroot, bb, p):
    return pl.pallas_call(
        _dense_body,
        grid=(GA,),
        in_specs=[
            pl.BlockSpec((NC, RB, D), lambda i: (0, i, 0)),
            pl.BlockSpec((RB, D), lambda i: (i, 0)),
            pl.BlockSpec((D, D), lambda i: (0, 0)),
            pl.BlockSpec((D, D), lambda i: (0, 0)),
            pl.BlockSpec((1, D), lambda i: (0, 0)),
            pl.BlockSpec((D, 1), lambda i: (0, 0)),
        ],
        out_specs=[pl.BlockSpec((RB, D), lambda i: (i, 0)),
                   pl.BlockSpec((RB, 1), lambda i: (i, 0))],
        out_shape=[jax.ShapeDtypeStruct((N, D), jnp.float32),
                   jax.ShapeDtypeStruct((N, 1), jnp.float32)],
    )(parts, xm, wr, wroot, bb, p)


def _sortable(s):
    ibits = lax.bitcast_convert_type(s, jnp.int32)
    return jnp.where(ibits < 0, ibits ^ jnp.int32(0x7FFFFFFF), ibits)


def _select_thresholds(k, s2, m2, prev_keys2):
    skey = _sortable(s2)
    r = lax.broadcasted_iota(jnp.int32, (SROWS, 128), 0)
    c = lax.broadcasted_iota(jnp.int32, (SROWS, 128), 1)
    gidx = r * 128 + c
    valid = (m2 > 0) & (gidx < N)

    eq = valid
    need = jnp.int32(k)
    ts = []
    for key_full in [skey] + prev_keys2:
        key = jnp.where(eq, key_full, jnp.int32(IMIN))

        def tbit(i, pu):
            bb = 31 - i
            trial = pu | (jnp.int32(1) << bb)
            thr = trial ^ jnp.int32(IMIN)
            cnt = jnp.sum((key >= thr).astype(jnp.int32))
            return jnp.where(cnt >= need, trial, pu)
        pu = lax.fori_loop(0, 32, tbit, jnp.int32(0))
        t = pu ^ jnp.int32(IMIN)
        ts.append(t)
        need = need - jnp.sum((eq & (key > t)).astype(jnp.int32))
        eq = eq & (key == t)

    def jbit(i, jj):
        bb = 13 - i
        trial = jj | (jnp.int32(1) << bb)
        g = jnp.sum((eq & (gidx < trial)).astype(jnp.int32))
        return jnp.where(g < need, trial, jj)
    jmax = lax.fori_loop(0, 14, jbit, jnp.int32(0))
    return skey, ts, jmax, need


def _reconstruct(ts, jmax, needf, valid, keys, gidx):
    eq = valid
    sel = jnp.zeros_like(valid)
    for key_full, t in zip(keys, ts):
        key = jnp.where(eq, key_full, jnp.int32(IMIN))
        sel = sel | (eq & (key > t))
        eq = eq & (key == t)
    return sel | (eq & (gidx <= jmax) & (needf > 0))


def _layerB_body(k, nprev, h_ref, s_ref, s2_ref, m2_ref, mcol_ref, *refs):
    prev2 = [refs[i][...] for i in range(nprev)]
    prevcol = [refs[nprev + i][...] for i in range(nprev)]
    (y_ref, nm2_ref, nmcol_ref, skey2_ref, skeycol_ref,
     rmax_ref, rsum_ref) = refs[2 * nprev:]

    s2 = s2_ref[...]
    skey2, ts, jmax, needf = _select_thresholds(k, s2, m2_ref[...], prev2)
    sel2 = _reconstruct(
        ts, jmax, needf,
        (m2_ref[...] > 0) & ((lax.broadcasted_iota(jnp.int32, (SROWS, 128), 0) * 128
                              + lax.broadcasted_iota(jnp.int32, (SROWS, 128), 1)) < N),
        [skey2] + prev2,
        lax.broadcasted_iota(jnp.int32, (SROWS, 128), 0) * 128
        + lax.broadcasted_iota(jnp.int32, (SROWS, 128), 1))

    s = s_ref[...]
    skeycol = _sortable(s)
    gcol = lax.broadcasted_iota(jnp.int32, (N, 1), 0)
    selcol = _reconstruct(ts, jmax, needf, mcol_ref[...] > 0,
                          [skeycol] + prevcol, gcol)
    nmc = selcol.astype(jnp.float32)

    h = h_ref[...]
    y = h * (s * nmc)
    rmax = jnp.max(jnp.where(selcol, y, -jnp.inf), axis=0, keepdims=True)
    rsum = jnp.sum(y, axis=0, keepdims=True)

    y_ref[...] = y
    nm2_ref[...] = sel2.astype(jnp.float32)
    nmcol_ref[...] = nmc
    skey2_ref[...] = skey2
    skeycol_ref[...] = skeycol
    rmax_ref[...] = rmax
    rsum_ref[...] = rsum


def _layerB(k, nprev, h, s, s2, m2, mcol, prev2, prevcol):
    return pl.pallas_call(
        functools.partial(_layerB_body, k, nprev),
        out_shape=[jax.ShapeDtypeStruct((N, D), jnp.float32),
                   jax.ShapeDtypeStruct((SROWS, 128), jnp.float32),
                   jax.ShapeDtypeStruct((N, 1), jnp.float32),
                   jax.ShapeDtypeStruct((SROWS, 128), jnp.int32),
                   jax.ShapeDtypeStruct((N, 1), jnp.int32),
                   jax.ShapeDtypeStruct((1, D), jnp.float32),
                   jax.ShapeDtypeStruct((1, D), jnp.float32)],
    )(h, s, s2, m2, mcol, *prev2, *prevcol)


def _layer3B_body(k, nprev, h_ref, s_ref, s2_ref, m2_ref, mcol_ref, *refs):
    prev2 = [refs[i][...] for i in range(nprev)]
    prevcol = [refs[nprev + i][...] for i in range(nprev)]
    (mx1, sm1, mx2, sm2, wa, wb, b1, w2, b2, w3, b3, out_ref) = refs[2 * nprev:]

    s2 = s2_ref[...]
    skey2, ts, jmax, needf = _select_thresholds(k, s2, m2_ref[...], prev2)
    s = s_ref[...]
    skeycol = _sortable(s)
    gcol = lax.broadcasted_iota(jnp.int32, (N, 1), 0)
    selcol = _reconstruct(ts, jmax, needf, mcol_ref[...] > 0,
                          [skeycol] + prevcol, gcol)
    nmc = selcol.astype(jnp.float32)

    h = h_ref[...]
    y = h * (s * nmc)
    rmax = jnp.max(jnp.where(selcol, y, -jnp.inf), axis=0, keepdims=True)
    rsum = jnp.sum(y, axis=0, keepdims=True)

    zmax = mx1[...] + mx2[...] + rmax
    zmean = sm1[...] / KS[0] + sm2[...] / KS[1] + rsum / KS[2]
    a = jnp.maximum(jnp.dot(zmax, wa[...], preferred_element_type=jnp.float32)
                    + jnp.dot(zmean, wb[...], preferred_element_type=jnp.float32)
                    + b1[...], 0.0)
    a = jnp.maximum(jnp.dot(a, w2[...], preferred_element_type=jnp.float32)
                    + b2[...], 0.0)
    lg = jnp.dot(a, w3[...], preferred_element_type=jnp.float32) + b3[...]
    mx = jnp.max(lg, axis=1, keepdims=True)
    e = jnp.exp(lg - mx)
    out_ref[...] = lg - mx - jnp.log(jnp.sum(e, axis=1, keepdims=True))


def _layer3B(k, nprev, h, s, s2, m2, mcol, prev2, prevcol, reads,
             L1W, L1b, L2W, L2b, L3W, L3b):
    (mx1, sm1), (mx2, sm2) = reads
    return pl.pallas_call(
        functools.partial(_layer3B_body, k, nprev),
        out_shape=jax.ShapeDtypeStruct((1, 16), jnp.float32),
    )(h, s, s2, m2, mcol, *prev2, *prevcol,
      mx1, sm1, mx2, sm2, L1W[:D], L1W[D:], L1b.reshape(1, -1),
      L2W, L2b.reshape(1, -1), L3W, L3b.reshape(1, -1))


def kernel(x, edge_index, batch, W1r, b1, W1root, p1, W2r, b2, W2root, p2,
           W3r, b3, W3root, p3, L1W, L1b, L2W, L2b, L3W, L3b):
    src = edge_index[0]
    dst = edge_index[1]
    Ws = ((W1r, b1, W1root, p1), (W2r, b2, W2root, p2), (W3r, b3, W3root, p3))

    xm = x
    m2 = jnp.ones((SROWS, 128), jnp.float32)
    mcol = jnp.ones((N, 1), jnp.float32)
    prev2, prevcol = [], []
    reads = []
    for l in range(2):
        Wr, bb, Wroot, p = Ws[l]
        parts = _sc_scatter(xm, src, dst)
        h, s = _dense(parts, xm, Wr, Wroot, bb.reshape(1, D), p.reshape(D, 1))
        s2 = jnp.reshape(jnp.pad(s, ((0, NPAD - N), (0, 0))), (SROWS, 128))
        (xm, m2, mcol, skey2, skeycol, rmax, rsum) = _layerB(
            KS[l], len(prev2), h, s, s2, m2, mcol, prev2, prevcol)
        prev2.insert(0, skey2)
        prevcol.insert(0, skeycol)
        reads.append((rmax, rsum))

    Wr, bb, Wroot, p = Ws[2]
    parts = _sc_scatter(xm, src, dst)
    h, s = _dense(parts, xm, Wr, Wroot, bb.reshape(1, D), p.reshape(D, 1))
    s2 = jnp.reshape(jnp.pad(s, ((0, NPAD - N), (0, 0))), (SROWS, 128))
    return _layer3B(KS[2], len(prev2), h, s, s2, m2, mcol, prev2, prevcol,
                    reads, L1W, L1b, L2W, L2b, L3W, L3b)
